# baseline (speedup 1.0000x reference)
import jax
import jax.numpy as jnp
from jax.experimental import pallas as pl
from jax.experimental.pallas import tpu as pltpu

FEAT = 256
NCLS = 4768
NQ = 2000
NTOT = NCLS + NQ
NPAD = 6784
SCALE = 30.0
B = 4096
BB = 256
NB = B // BB


def _oim_body(x_ref, wt_ref, t_ref, logits_ref, loss_ref):
    i = pl.program_id(0)
    x = x_ref[...]
    nrm = jnp.sqrt(jnp.sum(x * x, axis=1, keepdims=True)) + 1e-12
    xn = x / nrm
    z = jax.lax.dot_general(
        xn.astype(jnp.bfloat16), wt_ref[...],
        (((1,), (0,)), ((), ())),
        preferred_element_type=jnp.float32,
    ) * SCALE
    logits_ref[...] = z
    cols = jax.lax.broadcasted_iota(jnp.int32, (BB, NPAD), 1)
    valid = cols < NTOT
    sumexp = jnp.sum(jnp.where(valid, jnp.exp(z - SCALE), 0.0), axis=1)
    t = t_ref[0, 0, :]
    tlogit = jnp.sum(jnp.where(cols == t[:, None], z, 0.0), axis=1)
    partial = jnp.sum(SCALE + jnp.log(sumexp) - tlogit) * (1.0 / B)

    @pl.when(i == 0)
    def _():
        loss_ref[0, 0] = 0.0

    loss_ref[0, 0] += partial


def kernel(inputs, targets, lut, queue):
    w = jnp.concatenate(
        [lut, queue, jnp.zeros((NPAD - NTOT, FEAT), jnp.float32)], axis=0
    )
    wt = w.T.astype(jnp.bfloat16)
    t3 = targets.reshape(NB, 1, BB)
    padded, loss = pl.pallas_call(
        _oim_body,
        grid=(NB,),
        in_specs=[
            pl.BlockSpec((BB, FEAT), lambda i: (i, 0)),
            pl.BlockSpec((FEAT, NPAD), lambda i: (0, 0)),
            pl.BlockSpec((1, 1, BB), lambda i: (i, 0, 0)),
        ],
        out_specs=[
            pl.BlockSpec((BB, NPAD), lambda i: (i, 0)),
            pl.BlockSpec(memory_space=pltpu.SMEM),
        ],
        out_shape=[
            jax.ShapeDtypeStruct((B, NPAD), jnp.float32),
            jax.ShapeDtypeStruct((1, 1), jnp.float32),
        ],
    )(inputs, wt, t3)
    return (loss[0, 0], padded[:, :NTOT])
